# baseline (speedup 1.0000x reference)
import jax
import jax.numpy as jnp
from jax.experimental import pallas as pl
from jax.experimental.pallas import tpu as pltpu

_DEPTH = 1000
_N = 16384
_BR = 256


def _onehot_block(idx_ref, out_ref):
    idx = idx_ref[...]
    cols = jax.lax.broadcasted_iota(jnp.int32, (_BR, _DEPTH), 1)
    out_ref[...] = jnp.where(cols == idx, jnp.float32(1.0), jnp.float32(0.0))


def kernel(inputs):
    idx = inputs.astype(jnp.int32).reshape(_N, 1)
    grid = _N // _BR
    return pl.pallas_call(
        _onehot_block,
        grid=(grid,),
        in_specs=[pl.BlockSpec((_BR, 1), lambda i: (i, 0))],
        out_specs=pl.BlockSpec((_BR, _DEPTH), lambda i: (i, 0)),
        out_shape=jax.ShapeDtypeStruct((_N, _DEPTH), jnp.float32),
        compiler_params=pltpu.CompilerParams(
            dimension_semantics=("arbitrary",),
        ),
    )(idx)


# speedup vs baseline: 1.2881x; 1.2881x over previous
import jax
import jax.numpy as jnp
from jax.experimental import pallas as pl
from jax.experimental.pallas import tpu as pltpu

_DEPTH = 1000
_N = 16384
_BR = 4096


def _onehot_block(idx_ref, out_ref):
    idx = idx_ref[...]
    cols = jax.lax.broadcasted_iota(jnp.int32, (_BR, _DEPTH), 1)
    out_ref[...] = jnp.where(cols == idx, jnp.float32(1.0), jnp.float32(0.0))


def kernel(inputs):
    idx = inputs.astype(jnp.int32).reshape(_N, 1)
    grid = _N // _BR
    return pl.pallas_call(
        _onehot_block,
        grid=(grid,),
        in_specs=[pl.BlockSpec((_BR, 1), lambda i: (i, 0))],
        out_specs=pl.BlockSpec((_BR, _DEPTH), lambda i: (i, 0)),
        out_shape=jax.ShapeDtypeStruct((_N, _DEPTH), jnp.float32),
        compiler_params=pltpu.CompilerParams(
            dimension_semantics=("arbitrary",),
        ),
    )(idx)
